# Initial kernel scaffold; baseline (speedup 1.0000x reference)
import jax
import jax.numpy as jnp
from jax.experimental import pallas as pl


def kernel(x, W1, b1, ln1_g, ln1_b, W2, b2, ln2_g, ln2_b, Wqkv, bqkv, Wo, bo, Wf1, bf1, Wf2, bf2):
    raise NotImplementedError("write your pallas kernel here")


# speedup vs baseline: 2.2073x; 2.2073x over previous
import functools
import jax
import jax.numpy as jnp
from jax import lax
from jax.experimental import pallas as pl
from jax.experimental.pallas import tpu as pltpu

L = 256
H = 8
DH = 16
DM = 128


def _ln(x, g, b):
    m = jnp.mean(x, axis=-1, keepdims=True)
    v = jnp.mean((x - m) ** 2, axis=-1, keepdims=True)
    return (x - m) * lax.rsqrt(v + 1e-5) * g + b


def _body(B, feats_ref, mask_ref,
          W1_ref, b1_ref, ln1g_ref, ln1b_ref,
          W2_ref, b2_ref, ln2g_ref, ln2b_ref,
          Wqkv_ref, bqkv_ref, Wo_ref, bo_ref,
          Wf1_ref, bf1_ref, Wf2_ref, bf2_ref,
          out_ref, qkv_ref, pooled_ref):
    f = feats_ref[...]
    h = jnp.dot(f, W1_ref[...], preferred_element_type=jnp.float32) + b1_ref[...]
    h = jax.nn.relu(_ln(h, ln1g_ref[...], ln1b_ref[...]))
    h = jnp.dot(h, W2_ref[...], preferred_element_type=jnp.float32) + b2_ref[...]
    h = jax.nn.relu(_ln(h, ln2g_ref[...], ln2b_ref[...]))
    qkv_ref[...] = (
        jnp.dot(h, Wqkv_ref[...], preferred_element_type=jnp.float32)
        + bqkv_ref[...]
    )

    def per_face(s, _):
        q = qkv_ref[pl.ds(s * L, L), 0:DM]
        k = qkv_ref[pl.ds(s * L, L), DM:2 * DM]
        v = qkv_ref[pl.ds(s * L, L), 2 * DM:3 * DM]
        mrow = mask_ref[pl.ds(s, 1), :]
        keypad = mrow == 0.0
        o_parts = []
        for hh in range(H):
            qh = q[:, hh * DH:(hh + 1) * DH]
            kh = k[:, hh * DH:(hh + 1) * DH]
            vh = v[:, hh * DH:(hh + 1) * DH]
            logits = lax.dot_general(
                qh, kh, (((1,), (1,)), ((), ())),
                preferred_element_type=jnp.float32) * 0.25
            logits = jnp.where(keypad, jnp.float32(-1e9), logits)
            mx = jnp.max(logits, axis=-1, keepdims=True)
            e = jnp.exp(logits - mx)
            ssum = jnp.sum(e, axis=-1, keepdims=True)
            o_parts.append(
                jnp.dot(e, vh, preferred_element_type=jnp.float32) / ssum)
        o = jnp.concatenate(o_parts, axis=1)
        o = jnp.dot(o, Wo_ref[...], preferred_element_type=jnp.float32) + bo_ref[...]
        pooled = lax.dot_general(
            mrow, o, (((1,), (0,)), ((), ())),
            preferred_element_type=jnp.float32)
        denom = jnp.sum(mrow, axis=-1, keepdims=True) + 1e-8
        pooled_ref[pl.ds(s, 1), :] = pooled / denom
        return 0

    lax.fori_loop(0, B, per_face, 0)

    p = pooled_ref[...]
    p = jax.nn.relu(
        jnp.dot(p, Wf1_ref[...], preferred_element_type=jnp.float32)
        + bf1_ref[...])
    out_ref[...] = (
        jnp.dot(p, Wf2_ref[...], preferred_element_type=jnp.float32)
        + bf2_ref[...])


def kernel(x, W1, b1, ln1_g, ln1_b, W2, b2, ln2_g, ln2_b,
           Wqkv, bqkv, Wo, bo, Wf1, bf1, Wf2, bf2):
    N = x.shape[0]
    B = 8
    x3 = x.reshape(N, L, 4)
    feats = x3[..., :3].reshape(N * L, 3)
    mask = x3[..., 3]

    row = lambda a: a.reshape(1, -1)
    weights = (W1, row(b1), row(ln1_g), row(ln1_b),
               W2, row(b2), row(ln2_g), row(ln2_b),
               Wqkv, row(bqkv), Wo, row(bo),
               Wf1, row(bf1), Wf2, row(bf2))

    wspecs = [pl.BlockSpec(w.shape, lambda i: (0, 0)) for w in weights]

    return pl.pallas_call(
        functools.partial(_body, B),
        grid=(N // B,),
        in_specs=[
            pl.BlockSpec((B * L, 3), lambda i: (i, 0)),
            pl.BlockSpec((B, L), lambda i: (i, 0)),
            *wspecs,
        ],
        out_specs=pl.BlockSpec((B, 32), lambda i: (i, 0)),
        out_shape=jax.ShapeDtypeStruct((N, 32), jnp.float32),
        scratch_shapes=[
            pltpu.VMEM((B * L, 3 * DM), jnp.float32),
            pltpu.VMEM((B, DM), jnp.float32),
        ],
        compiler_params=pltpu.CompilerParams(
            dimension_semantics=("parallel",),
        ),
    )(feats, mask, *weights)
